# baseline (speedup 1.0000x reference)
import jax
import jax.numpy as jnp
from jax.experimental import pallas as pl

_CODE_DIM = 32
_NUM_CODES = 8192
_TOKEN_BLOCK = 1024


def _normalize_kernel(x_ref, out_ref):
    x = x_ref[...]
    out_ref[...] = x / jnp.maximum(
        jnp.sqrt(jnp.sum(x * x, axis=1, keepdims=True)), 1e-8)


def _vq_kernel(xn_ref, cbn_ref, out_ref):
    logits = jax.lax.dot_general(
        xn_ref[...], cbn_ref[...], (((1,), (1,)), ((), ())),
        preferred_element_type=jnp.float32)
    out_ref[0, 0, :] = jnp.argmax(logits, axis=1).astype(jnp.int32)


def kernel(z_e, codebook):
    b, t, d = z_e.shape
    n_tokens = b * t
    flat = z_e.reshape(n_tokens, d)
    n_blocks = n_tokens // _TOKEN_BLOCK

    stacked = jnp.concatenate([flat, codebook], axis=0)
    normed = pl.pallas_call(
        _normalize_kernel,
        out_shape=jax.ShapeDtypeStruct(stacked.shape, jnp.float32),
    )(stacked)
    xn = normed[:n_tokens]
    cbn = normed[n_tokens:]

    out = pl.pallas_call(
        _vq_kernel,
        grid=(n_blocks,),
        in_specs=[
            pl.BlockSpec((_TOKEN_BLOCK, _CODE_DIM), lambda i: (i, 0)),
            pl.BlockSpec((_NUM_CODES, _CODE_DIM), lambda i: (0, 0)),
        ],
        out_specs=pl.BlockSpec((1, 1, _TOKEN_BLOCK), lambda i: (i, 0, 0)),
        out_shape=jax.ShapeDtypeStruct((n_blocks, 1, _TOKEN_BLOCK), jnp.int32),
    )(xn, cbn)
    return out.reshape(b, t)
